# baseline (speedup 1.0000x reference)
import functools

import jax
import jax.numpy as jnp
from jax import lax
from jax.experimental import pallas as pl
from jax.experimental.pallas import tpu as pltpu
from jax.experimental.pallas import tpu_sc as plsc

L = 16
NW = 32
SLOTS = 8
BIG = 1e10
SENT = 1e5
PAD = 2 * L


_GDN = lax.GatherDimensionNumbers(
    offset_dims=(), collapsed_slice_dims=(0,), start_index_map=(0,))


def _lane_splat(v, uv):
    return lax.gather(v, uv[:, None], _GDN, (1,),
                      mode=lax.GatherScatterMode.PROMISE_IN_BOUNDS)


def _sc_chamfer(B, N):
    mesh = plsc.VectorSubcoreMesh(core_axis_name="c", subcore_axis_name="s")
    NCH = N // L

    @functools.partial(
        pl.kernel,
        mesh=mesh,
        out_type=jax.ShapeDtypeStruct((NW * L,), jnp.float32),
        scratch_types=[
            pltpu.VMEM((N,), jnp.float32),
            pltpu.VMEM((N,), jnp.float32),
            pltpu.VMEM((N,), jnp.float32),
            pltpu.VMEM((N,), jnp.float32),
            pltpu.VMEM((N,), jnp.float32),
            pltpu.VMEM((N,), jnp.float32),
            pltpu.VMEM((N,), jnp.int32),
            pltpu.VMEM((N + PAD,), jnp.float32),
            pltpu.VMEM((N + PAD,), jnp.float32),
            pltpu.VMEM((N + PAD,), jnp.float32),
            pltpu.VMEM((N + PAD,), jnp.float32),
            pltpu.VMEM((N + PAD,), jnp.float32),
            pltpu.VMEM((N + PAD,), jnp.float32),
            pltpu.VMEM((L,), jnp.float32),
        ],
        compiler_params=pltpu.CompilerParams(needs_layout_passes=False),
    )
    def cham(z_hbm, ax_hbm, ay_hbm, gx_hbm, gy_hbm, gz_hbm, m_hbm, out_hbm,
             z_v, ax_v, ay_v, gx_v, gy_v, gz_v, m_v,
             cpx, cpy, cpz, cgx, cgy, cgz, acc_v):
        cid = lax.axis_index("c")
        sid = lax.axis_index("s")
        wid = sid * 2 + cid
        bat = wid // SLOTS
        slot = wid % SLOTS
        boff = bat * N

        pltpu.sync_copy(z_hbm.at[pl.ds(boff, N)], z_v)
        pltpu.sync_copy(ax_hbm, ax_v)
        pltpu.sync_copy(ay_hbm, ay_v)
        pltpu.sync_copy(gx_hbm.at[pl.ds(boff, N)], gx_v)
        pltpu.sync_copy(gy_hbm.at[pl.ds(boff, N)], gy_v)
        pltpu.sync_copy(gz_hbm.at[pl.ds(boff, N)], gz_v)
        pltpu.sync_copy(m_hbm.at[pl.ds(boff, N)], m_v)

        def comp_body(i, carry):
            n_p, n_g = carry
            sl = pl.ds(i * L, L)
            zc = z_v[sl]
            pxc = ax_v[sl] * zc
            pyc = ay_v[sl] * zc
            gxc = gx_v[sl]
            gyc = gy_v[sl]
            gzc = gz_v[sl]
            mc = m_v[sl] > 0
            mp = mc & (pxc + pyc + zc != 0.0)
            mg = mc & (gxc + gyc + gzc != 0.0)
            mpi = mp.astype(jnp.int32)
            mgi = mg.astype(jnp.int32)
            pidx = n_p + (plsc.cumsum(mpi) - mpi)
            gidx = n_g + (plsc.cumsum(mgi) - mgi)
            plsc.store_scatter(cpx, [pidx], pxc, mask=mp)
            plsc.store_scatter(cpy, [pidx], pyc, mask=mp)
            plsc.store_scatter(cpz, [pidx], zc, mask=mp)
            plsc.store_scatter(cgx, [gidx], gxc, mask=mg)
            plsc.store_scatter(cgy, [gidx], gyc, mask=mg)
            plsc.store_scatter(cgz, [gidx], gzc, mask=mg)
            return (n_p + jnp.sum(mpi), n_g + jnp.sum(mgi))

        n_p, n_g = lax.fori_loop(0, NCH, comp_body,
                                 (jnp.int32(0), jnp.int32(0)))

        sent = jnp.full((L,), SENT, jnp.float32)
        cpx[pl.ds(n_p, L)] = sent
        cpy[pl.ds(n_p, L)] = sent
        cpz[pl.ds(n_p, L)] = sent
        cgx[pl.ds(n_g, L)] = sent
        cgy[pl.ds(n_g, L)] = sent
        cgz[pl.ds(n_g, L)] = sent

        lane = lax.iota(jnp.int32, L)
        _SPLATS = [jnp.full((L,), u, jnp.int32) for u in range(L)]

        def direction(qx_r, qy_r, qz_r, nq, rx_r, ry_r, rz_r, nr, acc):
            qper = (nq + SLOTS - 1) // SLOTS
            qlo = slot * qper
            qhi = jnp.minimum(nq, qlo + qper)
            nblk = (jnp.maximum(0, qhi - qlo) + L - 1) // L
            nrg = (nr + L - 1) // L

            def qblock(ib, acc):
                base = qlo + ib * L
                qx = qx_r[pl.ds(base, L)]
                qy = qy_r[pl.ds(base, L)]
                qz = qz_r[pl.ds(base, L)]

                def rloop(g, rmin):
                    goff = g * L
                    gx16 = rx_r[pl.ds(goff, L)]
                    gy16 = ry_r[pl.ds(goff, L)]
                    gz16 = rz_r[pl.ds(goff, L)]
                    for u in range(L):
                        uv = _SPLATS[u]
                        rx = _lane_splat(gx16, uv)
                        ry = _lane_splat(gy16, uv)
                        rz = _lane_splat(gz16, uv)
                        dx = qx - rx
                        dy = qy - ry
                        dz = qz - rz
                        d = dx * dx + dy * dy + dz * dz
                        rmin = jnp.minimum(rmin, d)
                    return rmin

                rmin = lax.fori_loop(0, nrg, rloop,
                                     jnp.full((L,), BIG, jnp.float32))
                valid = (base + lane) < qhi
                return acc + jnp.where(valid, rmin, 0.0)

            return lax.fori_loop(0, nblk, qblock, acc)

        acc = jnp.zeros((L,), jnp.float32)
        acc = direction(cpx, cpy, cpz, n_p, cgx, cgy, cgz, n_g, acc)
        acc = direction(cgx, cgy, cgz, n_g, cpx, cpy, cpz, n_p, acc)

        acc_v[...] = acc
        pltpu.sync_copy(acc_v, out_hbm.at[pl.ds(wid * L, L)])

    return cham


def kernel(pred, gt_xyz, mask, fx, fy, cx, cy):
    B, _, H, W = pred.shape
    N = H * W
    fx = jnp.asarray(fx, jnp.float32)
    fy = jnp.asarray(fy, jnp.float32)
    cx = jnp.asarray(cx, jnp.float32)
    cy = jnp.asarray(cy, jnp.float32)

    z = pred.reshape(B * N).astype(jnp.float32)
    gx = gt_xyz[:, 0, :, :].reshape(B * N).astype(jnp.float32)
    gy = gt_xyz[:, 1, :, :].reshape(B * N).astype(jnp.float32)
    gz = gt_xyz[:, 2, :, :].reshape(B * N).astype(jnp.float32)
    m = mask.reshape(B * N).astype(jnp.int32)
    n = jnp.arange(N, dtype=jnp.int32)
    ax = ((n % W).astype(jnp.float32) - cx) / fx
    ay = ((n // W).astype(jnp.float32) - cy) / fy

    out = _sc_chamfer(B, N)(z, ax, ay, gx, gy, gz, m)
    return jnp.sum(out) / jnp.float32(B)


# speedup vs baseline: 1.1518x; 1.1518x over previous
import functools

import jax
import jax.numpy as jnp
from jax import lax
from jax.experimental import pallas as pl
from jax.experimental.pallas import tpu as pltpu
from jax.experimental.pallas import tpu_sc as plsc

L = 16
NW = 32
SLOTS = 8
BIG = 1e10
SENT = 1e5
PAD = 2 * L
R_SPLIT = 2816
RB = 256


def _sc_chamfer(B, N):
    mesh = plsc.VectorSubcoreMesh(core_axis_name="c", subcore_axis_name="s")
    NP = N - R_SPLIT
    QS = N // SLOTS

    @functools.partial(
        pl.kernel,
        mesh=mesh,
        out_type=(jax.ShapeDtypeStruct((NW * L,), jnp.float32),
                  jax.ShapeDtypeStruct((B * N,), jnp.float32)),
        scratch_types=[
            pltpu.VMEM((N,), jnp.float32),
            pltpu.VMEM((N,), jnp.float32),
            pltpu.VMEM((N,), jnp.float32),
            pltpu.VMEM((N,), jnp.float32),
            pltpu.VMEM((N,), jnp.float32),
            pltpu.VMEM((N,), jnp.float32),
            pltpu.VMEM((N,), jnp.int32),
            pltpu.VMEM((NP + PAD,), jnp.float32),
            pltpu.VMEM((NP + PAD,), jnp.float32),
            pltpu.VMEM((NP + PAD,), jnp.float32),
            pltpu.VMEM((N + PAD,), jnp.float32),
            pltpu.VMEM((N + PAD,), jnp.float32),
            pltpu.VMEM((N + PAD,), jnp.float32),
            pltpu.VMEM((L,), jnp.float32),
            pltpu.VMEM((QS,), jnp.float32),
        ],
        compiler_params=pltpu.CompilerParams(needs_layout_passes=False),
    )
    def cham(z_hbm, ax_hbm, ay_hbm, gx_hbm, gy_hbm, gz_hbm, m_hbm,
             out1_hbm, out2_hbm,
             z_v, ax_v, ay_v, gx_v, gy_v, gz_v, m_v,
             cpx, cpy, cpz, cgx, cgy, cgz, acc_v, minb_v):
        cid = lax.axis_index("c")
        sid = lax.axis_index("s")
        wid = sid * 2 + cid
        bat = wid // SLOTS
        slot = wid % SLOTS
        boff = bat * N

        pltpu.sync_copy(z_hbm.at[pl.ds(boff, N)], z_v)
        pltpu.sync_copy(ax_hbm, ax_v)
        pltpu.sync_copy(ay_hbm, ay_v)
        pltpu.sync_copy(gx_hbm.at[pl.ds(boff, N)], gx_v)
        pltpu.sync_copy(gy_hbm.at[pl.ds(boff, N)], gy_v)
        pltpu.sync_copy(gz_hbm.at[pl.ds(boff, N)], gz_v)
        pltpu.sync_copy(m_hbm.at[pl.ds(boff, N)], m_v)

        def comp_g(i, n_g):
            sl = pl.ds(i * L, L)
            gxc = gx_v[sl]
            gyc = gy_v[sl]
            gzc = gz_v[sl]
            mg = (m_v[sl] > 0) & (gxc + gyc + gzc != 0.0)
            mgi = mg.astype(jnp.int32)
            gidx = n_g + (plsc.cumsum(mgi) - mgi)
            plsc.store_scatter(cgx, [gidx], gxc, mask=mg)
            plsc.store_scatter(cgy, [gidx], gyc, mask=mg)
            plsc.store_scatter(cgz, [gidx], gzc, mask=mg)
            return n_g + jnp.sum(mgi)

        def comp_p(i, n_p):
            sl = pl.ds(R_SPLIT + i * L, L)
            zc = z_v[sl]
            pxc = ax_v[sl] * zc
            pyc = ay_v[sl] * zc
            mp = (m_v[sl] > 0) & (pxc + pyc + zc != 0.0)
            mpi = mp.astype(jnp.int32)
            pidx = n_p + (plsc.cumsum(mpi) - mpi)
            plsc.store_scatter(cpx, [pidx], pxc, mask=mp)
            plsc.store_scatter(cpy, [pidx], pyc, mask=mp)
            plsc.store_scatter(cpz, [pidx], zc, mask=mp)
            return n_p + jnp.sum(mpi)

        n_g = lax.fori_loop(0, N // L, comp_g, jnp.int32(0))
        n_p = lax.fori_loop(0, NP // L, comp_p, jnp.int32(0))

        sent = jnp.full((L,), SENT, jnp.float32)
        cpx[pl.ds(n_p, L)] = sent
        cpy[pl.ds(n_p, L)] = sent
        cpz[pl.ds(n_p, L)] = sent
        cgx[pl.ds(n_g, L)] = sent
        cgy[pl.ds(n_g, L)] = sent
        cgz[pl.ds(n_g, L)] = sent

        lane = lax.iota(jnp.int32, L)

        def nn_min(qx, qy, qz, rx_r, ry_r, rz_r, nr4):
            def rloop(j, rmin):
                j4 = j * 4
                for u in range(4):
                    jv = jnp.full((L,), j4 + u, jnp.int32)
                    rx = plsc.load_gather(rx_r, [jv])
                    ry = plsc.load_gather(ry_r, [jv])
                    rz = plsc.load_gather(rz_r, [jv])
                    dx = qx - rx
                    dy = qy - ry
                    dz = qz - rz
                    d = dx * dx + dy * dy + dz * dz
                    rmin = jnp.minimum(rmin, d)
                return rmin

            return lax.fori_loop(0, nr4, rloop,
                                 jnp.full((L,), BIG, jnp.float32))

        qper = (n_p + SLOTS - 1) // SLOTS
        qlo = slot * qper
        qhi = jnp.minimum(n_p, qlo + qper)
        nblk = (jnp.maximum(0, qhi - qlo) + L - 1) // L
        ng4 = (n_g + 3) // 4

        def qblock(ib, acc):
            base = qlo + ib * L
            rmin = nn_min(cpx[pl.ds(base, L)], cpy[pl.ds(base, L)],
                          cpz[pl.ds(base, L)], cgx, cgy, cgz, ng4)
            valid = (base + lane) < qhi
            return acc + jnp.where(valid, rmin, 0.0)

        acc = lax.fori_loop(0, nblk, qblock, jnp.zeros((L,), jnp.float32))
        acc_v[...] = acc
        pltpu.sync_copy(acc_v, out1_hbm.at[pl.ds(wid * L, L)])

        np4 = (n_p + 3) // 4
        gbase = slot * QS

        def gblock(ib, _):
            off = ib * L
            sl = pl.ds(gbase + off, L)
            rmin = nn_min(gx_v[sl], gy_v[sl], gz_v[sl],
                          cpx, cpy, cpz, np4)
            minb_v[pl.ds(off, L)] = rmin
            return 0

        lax.fori_loop(0, QS // L, gblock, 0)
        pltpu.sync_copy(minb_v, out2_hbm.at[pl.ds(boff + gbase, QS)])

    return cham


def _tc_chamfer(B, N):
    NRB = R_SPLIT // RB

    def body(zt_ref, axt_ref, ayt_ref, mt_ref, gx_ref, gy_ref, gz_ref,
             mr_ref, out1_ref, out2_ref, colmin_s, acc_s):
        b = pl.program_id(0)
        r = pl.program_id(1)

        @pl.when((b == 0) & (r == 0))
        def _():
            acc_s[0, 0] = 0.0

        @pl.when(r == 0)
        def _():
            colmin_s[...] = jnp.full((1, N), BIG, jnp.float32)

        z = zt_ref[...]
        px = axt_ref[...] * z
        py = ayt_ref[...] * z
        gx = gx_ref[0]
        gy = gy_ref[0]
        gz = gz_ref[0]
        mp = (mt_ref[...] > 0) & (px + py + z != 0.0)
        mq = (mr_ref[0] > 0) & (gx + gy + gz != 0.0)

        dx = px - gx
        dy = py - gy
        dz = z - gz
        d = dx * dx + dy * dy + dz * dz

        dq = jnp.where(mq, d, BIG)
        rowmin = jnp.min(dq, axis=1, keepdims=True)
        acc_s[0, 0] += jnp.sum(jnp.where(mp, rowmin, 0.0))

        dp = jnp.where(mp, d, BIG)
        colmin_s[...] = jnp.minimum(colmin_s[...],
                                    jnp.min(dp, axis=0, keepdims=True))

        @pl.when(r == NRB - 1)
        def _():
            out2_ref[0] = colmin_s[...]

        @pl.when((b == B - 1) & (r == NRB - 1))
        def _():
            out1_ref[0, 0] = acc_s[0, 0]

    return pl.pallas_call(
        body,
        grid=(B, NRB),
        in_specs=[
            pl.BlockSpec((RB, 1), lambda b, r: (b * (N // RB) + r, 0)),
            pl.BlockSpec((RB, 1), lambda b, r: (r, 0)),
            pl.BlockSpec((RB, 1), lambda b, r: (r, 0)),
            pl.BlockSpec((RB, 1), lambda b, r: (b * (N // RB) + r, 0)),
            pl.BlockSpec((1, 1, N), lambda b, r: (b, 0, 0)),
            pl.BlockSpec((1, 1, N), lambda b, r: (b, 0, 0)),
            pl.BlockSpec((1, 1, N), lambda b, r: (b, 0, 0)),
            pl.BlockSpec((1, 1, N), lambda b, r: (b, 0, 0)),
        ],
        out_specs=[
            pl.BlockSpec(memory_space=pltpu.SMEM),
            pl.BlockSpec((1, 1, N), lambda b, r: (b, 0, 0)),
        ],
        out_shape=[
            jax.ShapeDtypeStruct((1, 1), jnp.float32),
            jax.ShapeDtypeStruct((B, 1, N), jnp.float32),
        ],
        scratch_shapes=[pltpu.VMEM((1, N), jnp.float32),
                        pltpu.SMEM((1, 1), jnp.float32)],
    )


def _combine(B, N):
    def body(tc2_ref, sc2_ref, gx_ref, gy_ref, gz_ref, m_ref, out_ref):
        gx = gx_ref[...]
        gy = gy_ref[...]
        gz = gz_ref[...]
        mq = (m_ref[...] > 0) & (gx + gy + gz != 0.0)
        cm = jnp.minimum(tc2_ref[...], sc2_ref[...])
        out_ref[0, 0] = jnp.sum(jnp.where(mq, cm, 0.0))

    return pl.pallas_call(
        body,
        out_shape=jax.ShapeDtypeStruct((1, 1), jnp.float32),
        out_specs=pl.BlockSpec(memory_space=pltpu.SMEM),
    )


def kernel(pred, gt_xyz, mask, fx, fy, cx, cy):
    B, _, H, W = pred.shape
    N = H * W
    fx = jnp.asarray(fx, jnp.float32)
    fy = jnp.asarray(fy, jnp.float32)
    cx = jnp.asarray(cx, jnp.float32)
    cy = jnp.asarray(cy, jnp.float32)

    z = pred.reshape(B * N).astype(jnp.float32)
    gxf = gt_xyz[:, 0, :, :].reshape(B * N).astype(jnp.float32)
    gyf = gt_xyz[:, 1, :, :].reshape(B * N).astype(jnp.float32)
    gzf = gt_xyz[:, 2, :, :].reshape(B * N).astype(jnp.float32)
    m = mask.reshape(B * N).astype(jnp.int32)
    n = jnp.arange(N, dtype=jnp.int32)
    ax = ((n % W).astype(jnp.float32) - cx) / fx
    ay = ((n // W).astype(jnp.float32) - cy) / fy

    sc1, sc2 = _sc_chamfer(B, N)(z, ax, ay, gxf, gyf, gzf, m)

    tc1, tc2 = _tc_chamfer(B, N)(
        z.reshape(B * N, 1), ax.reshape(N, 1), ay.reshape(N, 1),
        m.reshape(B * N, 1),
        gxf.reshape(B, 1, N), gyf.reshape(B, 1, N), gzf.reshape(B, 1, N),
        m.reshape(B, 1, N))

    d2 = _combine(B, N)(tc2.reshape(B, N), sc2.reshape(B, N),
                        gxf.reshape(B, N), gyf.reshape(B, N),
                        gzf.reshape(B, N), m.reshape(B, N))

    total = jnp.sum(sc1) + tc1[0, 0] + d2[0, 0]
    return total / jnp.float32(B)


# speedup vs baseline: 1.7548x; 1.5235x over previous
import functools

import jax
import jax.numpy as jnp
from jax import lax
from jax.experimental import pallas as pl
from jax.experimental.pallas import tpu as pltpu
from jax.experimental.pallas import tpu_sc as plsc

L = 16
NW = 32
SLOTS = 8
BIG = 1e10
SENT = 1e5
PAD = 2 * L
R_SPLIT = 2816
RB = 256


def _sc_chamfer(B, N):
    mesh = plsc.VectorSubcoreMesh(core_axis_name="c", subcore_axis_name="s")
    NP = N - R_SPLIT
    QS = N // SLOTS

    @functools.partial(
        pl.kernel,
        mesh=mesh,
        out_type=(jax.ShapeDtypeStruct((NW * L,), jnp.float32),
                  jax.ShapeDtypeStruct((B * N,), jnp.float32)),
        scratch_types=[
            pltpu.VMEM((N,), jnp.float32),
            pltpu.VMEM((N,), jnp.float32),
            pltpu.VMEM((N,), jnp.float32),
            pltpu.VMEM((N,), jnp.float32),
            pltpu.VMEM((N,), jnp.float32),
            pltpu.VMEM((N,), jnp.float32),
            pltpu.VMEM((N,), jnp.int32),
            pltpu.VMEM((NP + PAD,), jnp.float32),
            pltpu.VMEM((NP + PAD,), jnp.float32),
            pltpu.VMEM((NP + PAD,), jnp.float32),
            pltpu.VMEM((N + PAD,), jnp.float32),
            pltpu.VMEM((N + PAD,), jnp.float32),
            pltpu.VMEM((N + PAD,), jnp.float32),
            pltpu.VMEM((L,), jnp.float32),
            pltpu.VMEM((QS,), jnp.float32),
        ],
        compiler_params=pltpu.CompilerParams(needs_layout_passes=False),
    )
    def cham(z_hbm, ax_hbm, ay_hbm, gx_hbm, gy_hbm, gz_hbm, m_hbm,
             out1_hbm, out2_hbm,
             z_v, ax_v, ay_v, gx_v, gy_v, gz_v, m_v,
             cpx, cpy, cpz, cgx, cgy, cgz, acc_v, minb_v):
        cid = lax.axis_index("c")
        sid = lax.axis_index("s")
        wid = sid * 2 + cid
        bat = wid // SLOTS
        slot = wid % SLOTS
        boff = bat * N

        pltpu.sync_copy(z_hbm.at[pl.ds(boff, N)], z_v)
        pltpu.sync_copy(ax_hbm, ax_v)
        pltpu.sync_copy(ay_hbm, ay_v)
        pltpu.sync_copy(gx_hbm.at[pl.ds(boff, N)], gx_v)
        pltpu.sync_copy(gy_hbm.at[pl.ds(boff, N)], gy_v)
        pltpu.sync_copy(gz_hbm.at[pl.ds(boff, N)], gz_v)
        pltpu.sync_copy(m_hbm.at[pl.ds(boff, N)], m_v)

        def comp_g(i, n_g):
            sl = pl.ds(i * L, L)
            gxc = gx_v[sl]
            gyc = gy_v[sl]
            gzc = gz_v[sl]
            mg = (m_v[sl] > 0) & (gxc + gyc + gzc != 0.0)
            mgi = mg.astype(jnp.int32)
            gidx = n_g + (plsc.cumsum(mgi) - mgi)
            plsc.store_scatter(cgx, [gidx], gxc, mask=mg)
            plsc.store_scatter(cgy, [gidx], gyc, mask=mg)
            plsc.store_scatter(cgz, [gidx], gzc, mask=mg)
            return n_g + jnp.sum(mgi)

        def comp_p(i, n_p):
            sl = pl.ds(R_SPLIT + i * L, L)
            zc = z_v[sl]
            pxc = ax_v[sl] * zc
            pyc = ay_v[sl] * zc
            mp = (m_v[sl] > 0) & (pxc + pyc + zc != 0.0)
            mpi = mp.astype(jnp.int32)
            pidx = n_p + (plsc.cumsum(mpi) - mpi)
            plsc.store_scatter(cpx, [pidx], pxc, mask=mp)
            plsc.store_scatter(cpy, [pidx], pyc, mask=mp)
            plsc.store_scatter(cpz, [pidx], zc, mask=mp)
            return n_p + jnp.sum(mpi)

        n_g = lax.fori_loop(0, N // L, comp_g, jnp.int32(0))
        n_p = lax.fori_loop(0, NP // L, comp_p, jnp.int32(0))

        sent = jnp.full((L,), SENT, jnp.float32)
        cpx[pl.ds(n_p, L)] = sent
        cpy[pl.ds(n_p, L)] = sent
        cpz[pl.ds(n_p, L)] = sent
        cgx[pl.ds(n_g, L)] = sent
        cgy[pl.ds(n_g, L)] = sent
        cgz[pl.ds(n_g, L)] = sent

        lane = lax.iota(jnp.int32, L)

        def nn_min(qx, qy, qz, rx_r, ry_r, rz_r, nr4):
            def rloop(j, rmin):
                j4 = j * 4
                for u in range(4):
                    jv = jnp.full((L,), j4 + u, jnp.int32)
                    rx = plsc.load_gather(rx_r, [jv])
                    ry = plsc.load_gather(ry_r, [jv])
                    rz = plsc.load_gather(rz_r, [jv])
                    dx = qx - rx
                    dy = qy - ry
                    dz = qz - rz
                    d = dx * dx + dy * dy + dz * dz
                    rmin = jnp.minimum(rmin, d)
                return rmin

            return lax.fori_loop(0, nr4, rloop,
                                 jnp.full((L,), BIG, jnp.float32))

        qper = (n_p + SLOTS - 1) // SLOTS
        qlo = slot * qper
        qhi = jnp.minimum(n_p, qlo + qper)
        nblk = (jnp.maximum(0, qhi - qlo) + L - 1) // L
        ng4 = (n_g + 3) // 4

        def qblock(ib, acc):
            base = qlo + ib * L
            rmin = nn_min(cpx[pl.ds(base, L)], cpy[pl.ds(base, L)],
                          cpz[pl.ds(base, L)], cgx, cgy, cgz, ng4)
            valid = (base + lane) < qhi
            return acc + jnp.where(valid, rmin, 0.0)

        acc = lax.fori_loop(0, nblk, qblock, jnp.zeros((L,), jnp.float32))
        acc_v[...] = acc
        pltpu.sync_copy(acc_v, out1_hbm.at[pl.ds(wid * L, L)])

        np4 = (n_p + 3) // 4
        gbase = slot * QS

        def gblock(ib, _):
            off = ib * L
            sl = pl.ds(gbase + off, L)
            rmin = nn_min(gx_v[sl], gy_v[sl], gz_v[sl],
                          cpx, cpy, cpz, np4)
            minb_v[pl.ds(off, L)] = rmin
            return 0

        lax.fori_loop(0, QS // L, gblock, 0)
        pltpu.sync_copy(minb_v, out2_hbm.at[pl.ds(boff + gbase, QS)])

    return cham


def _tc_chamfer(B, N):
    NRB = R_SPLIT // RB

    def body(z_ref, ax_ref, ay_ref, mp_ref, gt_ref, mr_ref,
             out1_ref, out2_ref, colmin_s, acc_s):
        b = pl.program_id(0)
        r = pl.program_id(1)

        @pl.when((b == 0) & (r == 0))
        def _():
            acc_s[0, 0] = 0.0

        @pl.when(r == 0)
        def _():
            colmin_s[...] = jnp.full((1, N), BIG, jnp.float32)

        z = z_ref[0]
        px = ax_ref[0] * z
        py = ay_ref[0] * z
        mp = (mp_ref[0] > 0) & (px + py + z != 0.0)
        px = jnp.where(mp, px, SENT)
        py = jnp.where(mp, py, SENT)
        pz = jnp.where(mp, z, SENT)

        gq = gt_ref[0]
        gx, gy, gz = gq[0:1], gq[1:2], gq[2:3]
        mq = (mr_ref[0] > 0) & (gx + gy + gz != 0.0)
        gx = jnp.where(mq, gx, SENT)
        gy = jnp.where(mq, gy, SENT)
        gz = jnp.where(mq, gz, SENT)

        pp = px * px + py * py + pz * pz
        qq = gx * gx + gy * gy + gz * gz
        one_p = jnp.ones((1, RB), jnp.float32)
        one_q = jnp.ones((1, N), jnp.float32)
        zer_p = jnp.zeros((3, RB), jnp.float32)
        zer_q = jnp.zeros((3, N), jnp.float32)

        pm = jnp.concatenate(
            [-2.0 * px, -2.0 * py, -2.0 * pz, pp, one_p, zer_p], axis=0)
        qm = jnp.concatenate([gx, gy, gz, one_q, qq, zer_q], axis=0)
        d = lax.dot_general(jnp.transpose(pm, (1, 0)), qm,
                            (((1,), (0,)), ((), ())),
                            preferred_element_type=jnp.float32)
        d = jnp.maximum(d, 0.0)

        rowmin = jnp.minimum(jnp.min(d, axis=1, keepdims=True), BIG)
        mpf = jnp.where(mp, 1.0, 0.0)
        s1 = lax.dot_general(mpf, rowmin, (((1,), (0,)), ((), ())),
                             preferred_element_type=jnp.float32)
        acc_s[0, 0] += s1[0, 0]

        colmin_s[...] = jnp.minimum(colmin_s[...],
                                    jnp.min(d, axis=0, keepdims=True))

        @pl.when(r == NRB - 1)
        def _():
            out2_ref[...] = jnp.minimum(colmin_s[...], BIG)[0]

        @pl.when((b == B - 1) & (r == NRB - 1))
        def _():
            out1_ref[0, 0] = acc_s[0, 0]

    return pl.pallas_call(
        body,
        grid=(B, NRB),
        in_specs=[
            pl.BlockSpec((1, 1, RB), lambda b, r: (b, 0, r)),
            pl.BlockSpec((1, 1, RB), lambda b, r: (0, 0, r)),
            pl.BlockSpec((1, 1, RB), lambda b, r: (0, 0, r)),
            pl.BlockSpec((1, 1, RB), lambda b, r: (b, 0, r)),
            pl.BlockSpec((1, 3, N), lambda b, r: (b, 0, 0)),
            pl.BlockSpec((1, 1, N), lambda b, r: (b, 0, 0)),
        ],
        out_specs=[
            pl.BlockSpec(memory_space=pltpu.SMEM),
            pl.BlockSpec((N,), lambda b, r: (b,)),
        ],
        out_shape=[
            jax.ShapeDtypeStruct((1, 1), jnp.float32),
            jax.ShapeDtypeStruct((B * N,), jnp.float32),
        ],
        scratch_shapes=[pltpu.VMEM((1, N), jnp.float32),
                        pltpu.SMEM((1, 1), jnp.float32)],
    )


def _combine(B, N):
    def body(tc2_ref, sc2_ref, gx_ref, gy_ref, gz_ref, m_ref, out_ref,
             acc_s):
        b = pl.program_id(0)

        @pl.when(b == 0)
        def _():
            acc_s[0, 0] = 0.0

        gx = gx_ref[...]
        gy = gy_ref[...]
        gz = gz_ref[...]
        mq = (m_ref[...] > 0) & (gx + gy + gz != 0.0)
        cm = jnp.minimum(tc2_ref[...], sc2_ref[...])
        acc_s[0, 0] += jnp.sum(jnp.where(mq, cm, 0.0))

        @pl.when(b == B - 1)
        def _():
            out_ref[0, 0] = acc_s[0, 0]

    vspec = pl.BlockSpec((N,), lambda b: (b,))
    return pl.pallas_call(
        body,
        grid=(B,),
        in_specs=[vspec] * 6,
        out_specs=pl.BlockSpec(memory_space=pltpu.SMEM),
        out_shape=jax.ShapeDtypeStruct((1, 1), jnp.float32),
        scratch_shapes=[pltpu.SMEM((1, 1), jnp.float32)],
    )


def kernel(pred, gt_xyz, mask, fx, fy, cx, cy):
    B, _, H, W = pred.shape
    N = H * W
    fx = jnp.asarray(fx, jnp.float32)
    fy = jnp.asarray(fy, jnp.float32)
    cx = jnp.asarray(cx, jnp.float32)
    cy = jnp.asarray(cy, jnp.float32)

    z = pred.reshape(B * N).astype(jnp.float32)
    gxf = gt_xyz[:, 0, :, :].reshape(B * N).astype(jnp.float32)
    gyf = gt_xyz[:, 1, :, :].reshape(B * N).astype(jnp.float32)
    gzf = gt_xyz[:, 2, :, :].reshape(B * N).astype(jnp.float32)
    m = mask.reshape(B * N).astype(jnp.int32)
    n = jnp.arange(N, dtype=jnp.int32)
    ax = ((n % W).astype(jnp.float32) - cx) / fx
    ay = ((n // W).astype(jnp.float32) - cy) / fy

    sc1, sc2 = _sc_chamfer(B, N)(z, ax, ay, gxf, gyf, gzf, m)

    tc1, tc2 = _tc_chamfer(B, N)(
        pred.reshape(B, 1, N).astype(jnp.float32),
        ax.reshape(1, 1, N), ay.reshape(1, 1, N),
        mask.reshape(B, 1, N).astype(jnp.int32),
        gt_xyz.reshape(B, 3, N).astype(jnp.float32),
        mask.reshape(B, 1, N).astype(jnp.int32))

    d2 = _combine(B, N)(tc2, sc2, gxf, gyf, gzf, m)

    total = jnp.sum(sc1) + tc1[0, 0] + d2[0, 0]
    return total / jnp.float32(B)


# speedup vs baseline: 2.2773x; 1.2978x over previous
import functools

import jax
import jax.numpy as jnp
from jax import lax
from jax.experimental import pallas as pl
from jax.experimental.pallas import tpu as pltpu
from jax.experimental.pallas import tpu_sc as plsc

L = 16
NW = 32
SLOTS = 8
BIG = 1e10
SENT = 1e5
PAD = 2 * L
R_SPLIT = 3328
RB = 256


def _sc_chamfer(B, N):
    mesh = plsc.VectorSubcoreMesh(core_axis_name="c", subcore_axis_name="s")
    NP = N - R_SPLIT
    QS = N // SLOTS

    @functools.partial(
        pl.kernel,
        mesh=mesh,
        out_type=(jax.ShapeDtypeStruct((NW * L,), jnp.float32),
                  jax.ShapeDtypeStruct((B * N,), jnp.float32)),
        scratch_types=[
            pltpu.VMEM((N,), jnp.float32),
            pltpu.VMEM((N,), jnp.float32),
            pltpu.VMEM((N,), jnp.float32),
            pltpu.VMEM((N,), jnp.float32),
            pltpu.VMEM((N,), jnp.float32),
            pltpu.VMEM((N,), jnp.float32),
            pltpu.VMEM((N,), jnp.int32),
            pltpu.VMEM((NP + PAD,), jnp.float32),
            pltpu.VMEM((NP + PAD,), jnp.float32),
            pltpu.VMEM((NP + PAD,), jnp.float32),
            pltpu.VMEM((N + PAD,), jnp.float32),
            pltpu.VMEM((N + PAD,), jnp.float32),
            pltpu.VMEM((N + PAD,), jnp.float32),
            pltpu.VMEM((L,), jnp.float32),
            pltpu.VMEM((QS,), jnp.float32),
        ],
        compiler_params=pltpu.CompilerParams(needs_layout_passes=False),
    )
    def cham(z_hbm, ax_hbm, ay_hbm, gx_hbm, gy_hbm, gz_hbm, m_hbm,
             out1_hbm, out2_hbm,
             z_v, ax_v, ay_v, gx_v, gy_v, gz_v, m_v,
             cpx, cpy, cpz, cgx, cgy, cgz, acc_v, minb_v):
        cid = lax.axis_index("c")
        sid = lax.axis_index("s")
        wid = sid * 2 + cid
        bat = wid // SLOTS
        slot = wid % SLOTS
        boff = bat * N

        pltpu.sync_copy(z_hbm.at[pl.ds(boff, N)], z_v)
        pltpu.sync_copy(ax_hbm, ax_v)
        pltpu.sync_copy(ay_hbm, ay_v)
        pltpu.sync_copy(gx_hbm.at[pl.ds(boff, N)], gx_v)
        pltpu.sync_copy(gy_hbm.at[pl.ds(boff, N)], gy_v)
        pltpu.sync_copy(gz_hbm.at[pl.ds(boff, N)], gz_v)
        pltpu.sync_copy(m_hbm.at[pl.ds(boff, N)], m_v)

        def comp_g(i, n_g):
            sl = pl.ds(i * L, L)
            gxc = gx_v[sl]
            gyc = gy_v[sl]
            gzc = gz_v[sl]
            mg = (m_v[sl] > 0) & (gxc + gyc + gzc != 0.0)
            mgi = mg.astype(jnp.int32)
            gidx = n_g + (plsc.cumsum(mgi) - mgi)
            plsc.store_scatter(cgx, [gidx], gxc, mask=mg)
            plsc.store_scatter(cgy, [gidx], gyc, mask=mg)
            plsc.store_scatter(cgz, [gidx], gzc, mask=mg)
            return n_g + jnp.sum(mgi)

        def comp_p(i, n_p):
            sl = pl.ds(R_SPLIT + i * L, L)
            zc = z_v[sl]
            pxc = ax_v[sl] * zc
            pyc = ay_v[sl] * zc
            mp = (m_v[sl] > 0) & (pxc + pyc + zc != 0.0)
            mpi = mp.astype(jnp.int32)
            pidx = n_p + (plsc.cumsum(mpi) - mpi)
            plsc.store_scatter(cpx, [pidx], pxc, mask=mp)
            plsc.store_scatter(cpy, [pidx], pyc, mask=mp)
            plsc.store_scatter(cpz, [pidx], zc, mask=mp)
            return n_p + jnp.sum(mpi)

        n_g = lax.fori_loop(0, N // L, comp_g, jnp.int32(0))
        n_p = lax.fori_loop(0, NP // L, comp_p, jnp.int32(0))

        sent = jnp.full((L,), SENT, jnp.float32)
        cpx[pl.ds(n_p, L)] = sent
        cpy[pl.ds(n_p, L)] = sent
        cpz[pl.ds(n_p, L)] = sent
        cgx[pl.ds(n_g, L)] = sent
        cgy[pl.ds(n_g, L)] = sent
        cgz[pl.ds(n_g, L)] = sent

        lane = lax.iota(jnp.int32, L)

        def nn_min(qx, qy, qz, rx_r, ry_r, rz_r, nr4):
            def rloop(j, rmin):
                j4 = j * 4
                for u in range(4):
                    jv = jnp.full((L,), j4 + u, jnp.int32)
                    rx = plsc.load_gather(rx_r, [jv])
                    ry = plsc.load_gather(ry_r, [jv])
                    rz = plsc.load_gather(rz_r, [jv])
                    dx = qx - rx
                    dy = qy - ry
                    dz = qz - rz
                    d = dx * dx + dy * dy + dz * dz
                    rmin = jnp.minimum(rmin, d)
                return rmin

            return lax.fori_loop(0, nr4, rloop,
                                 jnp.full((L,), BIG, jnp.float32))

        qper = (n_p + SLOTS - 1) // SLOTS
        qlo = slot * qper
        qhi = jnp.minimum(n_p, qlo + qper)
        nblk = (jnp.maximum(0, qhi - qlo) + L - 1) // L
        ng4 = (n_g + 3) // 4

        def qblock(ib, acc):
            base = qlo + ib * L
            rmin = nn_min(cpx[pl.ds(base, L)], cpy[pl.ds(base, L)],
                          cpz[pl.ds(base, L)], cgx, cgy, cgz, ng4)
            valid = (base + lane) < qhi
            return acc + jnp.where(valid, rmin, 0.0)

        acc = lax.fori_loop(0, nblk, qblock, jnp.zeros((L,), jnp.float32))
        acc_v[...] = acc
        pltpu.sync_copy(acc_v, out1_hbm.at[pl.ds(wid * L, L)])

        np4 = (n_p + 3) // 4
        gbase = slot * QS

        def gblock(ib, _):
            off = ib * L
            sl = pl.ds(gbase + off, L)
            rmin = nn_min(gx_v[sl], gy_v[sl], gz_v[sl],
                          cpx, cpy, cpz, np4)
            minb_v[pl.ds(off, L)] = rmin
            return 0

        lax.fori_loop(0, QS // L, gblock, 0)
        pltpu.sync_copy(minb_v, out2_hbm.at[pl.ds(boff + gbase, QS)])

    return cham


def _tc_chamfer(B, N):
    NRB = R_SPLIT // RB

    def body(z_ref, ax_ref, ay_ref, mp_ref, gt_ref, mr_ref,
             out1_ref, out2_ref, colmin_s, acc_s):
        b = pl.program_id(0)
        r = pl.program_id(1)

        @pl.when((b == 0) & (r == 0))
        def _():
            acc_s[0, 0] = 0.0

        @pl.when(r == 0)
        def _():
            colmin_s[...] = jnp.full((1, N), BIG, jnp.float32)

        z = z_ref[0]
        px = ax_ref[0] * z
        py = ay_ref[0] * z
        mp = (mp_ref[0] > 0) & (px + py + z != 0.0)
        px = jnp.where(mp, px, SENT)
        py = jnp.where(mp, py, SENT)
        pz = jnp.where(mp, z, SENT)

        gq = gt_ref[0]
        gx, gy, gz = gq[0:1], gq[1:2], gq[2:3]
        mq = (mr_ref[0] > 0) & (gx + gy + gz != 0.0)
        gx = jnp.where(mq, gx, SENT)
        gy = jnp.where(mq, gy, SENT)
        gz = jnp.where(mq, gz, SENT)

        pp = px * px + py * py + pz * pz
        qq = gx * gx + gy * gy + gz * gz
        one_p = jnp.ones((1, RB), jnp.float32)
        one_q = jnp.ones((1, N), jnp.float32)
        zer_p = jnp.zeros((3, RB), jnp.float32)
        zer_q = jnp.zeros((3, N), jnp.float32)

        pm = jnp.concatenate(
            [-2.0 * px, -2.0 * py, -2.0 * pz, pp, one_p, zer_p], axis=0)
        qm = jnp.concatenate([gx, gy, gz, one_q, qq, zer_q], axis=0)
        d = lax.dot_general(jnp.transpose(pm, (1, 0)), qm,
                            (((1,), (0,)), ((), ())),
                            preferred_element_type=jnp.float32)
        d = jnp.maximum(d, 0.0)

        rowmin = jnp.minimum(jnp.min(d, axis=1, keepdims=True), BIG)
        mpf = jnp.where(mp, 1.0, 0.0)
        s1 = lax.dot_general(mpf, rowmin, (((1,), (0,)), ((), ())),
                             preferred_element_type=jnp.float32)
        acc_s[0, 0] += s1[0, 0]

        colmin_s[...] = jnp.minimum(colmin_s[...],
                                    jnp.min(d, axis=0, keepdims=True))

        @pl.when(r == NRB - 1)
        def _():
            out2_ref[...] = jnp.minimum(colmin_s[...], BIG)[0]

        @pl.when((b == B - 1) & (r == NRB - 1))
        def _():
            out1_ref[0, 0] = acc_s[0, 0]

    return pl.pallas_call(
        body,
        grid=(B, NRB),
        in_specs=[
            pl.BlockSpec((1, 1, RB), lambda b, r: (b, 0, r)),
            pl.BlockSpec((1, 1, RB), lambda b, r: (0, 0, r)),
            pl.BlockSpec((1, 1, RB), lambda b, r: (0, 0, r)),
            pl.BlockSpec((1, 1, RB), lambda b, r: (b, 0, r)),
            pl.BlockSpec((1, 3, N), lambda b, r: (b, 0, 0)),
            pl.BlockSpec((1, 1, N), lambda b, r: (b, 0, 0)),
        ],
        out_specs=[
            pl.BlockSpec(memory_space=pltpu.SMEM),
            pl.BlockSpec((N,), lambda b, r: (b,)),
        ],
        out_shape=[
            jax.ShapeDtypeStruct((1, 1), jnp.float32),
            jax.ShapeDtypeStruct((B * N,), jnp.float32),
        ],
        scratch_shapes=[pltpu.VMEM((1, N), jnp.float32),
                        pltpu.SMEM((1, 1), jnp.float32)],
    )


def _combine(B, N):
    def body(tc2_ref, sc2_ref, gx_ref, gy_ref, gz_ref, m_ref, out_ref,
             acc_s):
        b = pl.program_id(0)

        @pl.when(b == 0)
        def _():
            acc_s[0, 0] = 0.0

        gx = gx_ref[...]
        gy = gy_ref[...]
        gz = gz_ref[...]
        mq = (m_ref[...] > 0) & (gx + gy + gz != 0.0)
        cm = jnp.minimum(tc2_ref[...], sc2_ref[...])
        acc_s[0, 0] += jnp.sum(jnp.where(mq, cm, 0.0))

        @pl.when(b == B - 1)
        def _():
            out_ref[0, 0] = acc_s[0, 0]

    vspec = pl.BlockSpec((N,), lambda b: (b,))
    return pl.pallas_call(
        body,
        grid=(B,),
        in_specs=[vspec] * 6,
        out_specs=pl.BlockSpec(memory_space=pltpu.SMEM),
        out_shape=jax.ShapeDtypeStruct((1, 1), jnp.float32),
        scratch_shapes=[pltpu.SMEM((1, 1), jnp.float32)],
    )


def kernel(pred, gt_xyz, mask, fx, fy, cx, cy):
    B, _, H, W = pred.shape
    N = H * W
    fx = jnp.asarray(fx, jnp.float32)
    fy = jnp.asarray(fy, jnp.float32)
    cx = jnp.asarray(cx, jnp.float32)
    cy = jnp.asarray(cy, jnp.float32)

    z = pred.reshape(B * N).astype(jnp.float32)
    gxf = gt_xyz[:, 0, :, :].reshape(B * N).astype(jnp.float32)
    gyf = gt_xyz[:, 1, :, :].reshape(B * N).astype(jnp.float32)
    gzf = gt_xyz[:, 2, :, :].reshape(B * N).astype(jnp.float32)
    m = mask.reshape(B * N).astype(jnp.int32)
    n = jnp.arange(N, dtype=jnp.int32)
    ax = ((n % W).astype(jnp.float32) - cx) / fx
    ay = ((n // W).astype(jnp.float32) - cy) / fy

    sc1, sc2 = _sc_chamfer(B, N)(z, ax, ay, gxf, gyf, gzf, m)

    tc1, tc2 = _tc_chamfer(B, N)(
        pred.reshape(B, 1, N).astype(jnp.float32),
        ax.reshape(1, 1, N), ay.reshape(1, 1, N),
        mask.reshape(B, 1, N).astype(jnp.int32),
        gt_xyz.reshape(B, 3, N).astype(jnp.float32),
        mask.reshape(B, 1, N).astype(jnp.int32))

    d2 = _combine(B, N)(tc2, sc2, gxf, gyf, gzf, m)

    total = jnp.sum(sc1) + tc1[0, 0] + d2[0, 0]
    return total / jnp.float32(B)


# speedup vs baseline: 2.2872x; 1.0043x over previous
import functools

import jax
import jax.numpy as jnp
from jax import lax
from jax.experimental import pallas as pl
from jax.experimental.pallas import tpu as pltpu
from jax.experimental.pallas import tpu_sc as plsc

L = 16
NW = 32
SLOTS = 8
BIG = 1e10
SENT = 1e5
PAD = 2 * L
R_SPLIT = 3328
RB = 256


def _sc_chamfer(B, N):
    mesh = plsc.VectorSubcoreMesh(core_axis_name="c", subcore_axis_name="s")
    NP = N - R_SPLIT
    QS = N // SLOTS

    @functools.partial(
        pl.kernel,
        mesh=mesh,
        out_type=(jax.ShapeDtypeStruct((NW * L,), jnp.float32),
                  jax.ShapeDtypeStruct((B * N,), jnp.float32)),
        scratch_types=[
            pltpu.VMEM((N,), jnp.float32),
            pltpu.VMEM((N,), jnp.float32),
            pltpu.VMEM((N,), jnp.float32),
            pltpu.VMEM((N,), jnp.float32),
            pltpu.VMEM((N,), jnp.float32),
            pltpu.VMEM((N,), jnp.float32),
            pltpu.VMEM((N,), jnp.int32),
            pltpu.VMEM((NP + PAD,), jnp.float32),
            pltpu.VMEM((NP + PAD,), jnp.float32),
            pltpu.VMEM((NP + PAD,), jnp.float32),
            pltpu.VMEM((N + PAD,), jnp.float32),
            pltpu.VMEM((N + PAD,), jnp.float32),
            pltpu.VMEM((N + PAD,), jnp.float32),
            pltpu.VMEM((L,), jnp.float32),
            pltpu.VMEM((QS,), jnp.float32),
        ],
        compiler_params=pltpu.CompilerParams(needs_layout_passes=False),
    )
    def cham(z_hbm, ax_hbm, ay_hbm, gx_hbm, gy_hbm, gz_hbm, m_hbm,
             out1_hbm, out2_hbm,
             z_v, ax_v, ay_v, gx_v, gy_v, gz_v, m_v,
             cpx, cpy, cpz, cgx, cgy, cgz, acc_v, minb_v):
        cid = lax.axis_index("c")
        sid = lax.axis_index("s")
        wid = sid * 2 + cid
        bat = wid // SLOTS
        slot = wid % SLOTS
        boff = bat * N

        pltpu.sync_copy(z_hbm.at[pl.ds(boff, N)], z_v)
        pltpu.sync_copy(ax_hbm, ax_v)
        pltpu.sync_copy(ay_hbm, ay_v)
        pltpu.sync_copy(gx_hbm.at[pl.ds(boff, N)], gx_v)
        pltpu.sync_copy(gy_hbm.at[pl.ds(boff, N)], gy_v)
        pltpu.sync_copy(gz_hbm.at[pl.ds(boff, N)], gz_v)
        pltpu.sync_copy(m_hbm.at[pl.ds(boff, N)], m_v)

        def comp_g(i, n_g):
            sl = pl.ds(i * L, L)
            gxc = gx_v[sl]
            gyc = gy_v[sl]
            gzc = gz_v[sl]
            mg = (m_v[sl] > 0) & (gxc + gyc + gzc != 0.0)
            mgi = mg.astype(jnp.int32)
            gidx = n_g + (plsc.cumsum(mgi) - mgi)
            plsc.store_scatter(cgx, [gidx], gxc, mask=mg)
            plsc.store_scatter(cgy, [gidx], gyc, mask=mg)
            plsc.store_scatter(cgz, [gidx], gzc, mask=mg)
            return n_g + jnp.sum(mgi)

        def comp_p(i, n_p):
            sl = pl.ds(R_SPLIT + i * L, L)
            zc = z_v[sl]
            pxc = ax_v[sl] * zc
            pyc = ay_v[sl] * zc
            mp = (m_v[sl] > 0) & (pxc + pyc + zc != 0.0)
            mpi = mp.astype(jnp.int32)
            pidx = n_p + (plsc.cumsum(mpi) - mpi)
            plsc.store_scatter(cpx, [pidx], pxc, mask=mp)
            plsc.store_scatter(cpy, [pidx], pyc, mask=mp)
            plsc.store_scatter(cpz, [pidx], zc, mask=mp)
            return n_p + jnp.sum(mpi)

        n_g = lax.fori_loop(0, N // L, comp_g, jnp.int32(0))
        n_p = lax.fori_loop(0, NP // L, comp_p, jnp.int32(0))

        sent = jnp.full((L,), SENT, jnp.float32)
        cpx[pl.ds(n_p, L)] = sent
        cpy[pl.ds(n_p, L)] = sent
        cpz[pl.ds(n_p, L)] = sent
        cgx[pl.ds(n_g, L)] = sent
        cgy[pl.ds(n_g, L)] = sent
        cgz[pl.ds(n_g, L)] = sent

        lane = lax.iota(jnp.int32, L)

        def nn_min(qx, qy, qz, rx_r, ry_r, rz_r, nr8):
            def rloop(j, rmin):
                j8 = j * 8
                for u in range(8):
                    jv = jnp.full((L,), j8 + u, jnp.int32)
                    rx = plsc.load_gather(rx_r, [jv])
                    ry = plsc.load_gather(ry_r, [jv])
                    rz = plsc.load_gather(rz_r, [jv])
                    dx = qx - rx
                    dy = qy - ry
                    dz = qz - rz
                    d = dx * dx + dy * dy + dz * dz
                    rmin = jnp.minimum(rmin, d)
                return rmin

            return lax.fori_loop(0, nr8, rloop,
                                 jnp.full((L,), BIG, jnp.float32))

        qper = (n_p + SLOTS - 1) // SLOTS
        qlo = slot * qper
        qhi = jnp.minimum(n_p, qlo + qper)
        nblk = (jnp.maximum(0, qhi - qlo) + L - 1) // L
        ng8 = (n_g + 7) // 8

        def qblock(ib, acc):
            base = qlo + ib * L
            rmin = nn_min(cpx[pl.ds(base, L)], cpy[pl.ds(base, L)],
                          cpz[pl.ds(base, L)], cgx, cgy, cgz, ng8)
            valid = (base + lane) < qhi
            return acc + jnp.where(valid, rmin, 0.0)

        acc = lax.fori_loop(0, nblk, qblock, jnp.zeros((L,), jnp.float32))
        acc_v[...] = acc
        pltpu.sync_copy(acc_v, out1_hbm.at[pl.ds(wid * L, L)])

        np8 = (n_p + 7) // 8
        gbase = slot * QS

        def gblock(ib, _):
            off = ib * L
            sl = pl.ds(gbase + off, L)
            rmin = nn_min(gx_v[sl], gy_v[sl], gz_v[sl],
                          cpx, cpy, cpz, np8)
            minb_v[pl.ds(off, L)] = rmin
            return 0

        lax.fori_loop(0, QS // L, gblock, 0)
        pltpu.sync_copy(minb_v, out2_hbm.at[pl.ds(boff + gbase, QS)])

    return cham


def _tc_chamfer(B, N):
    NRB = R_SPLIT // RB

    def body(z_ref, ax_ref, ay_ref, mp_ref, gt_ref, mr_ref,
             out1_ref, out2_ref, colmin_s, acc_s):
        b = pl.program_id(0)
        r = pl.program_id(1)

        @pl.when((b == 0) & (r == 0))
        def _():
            acc_s[0, 0] = 0.0

        @pl.when(r == 0)
        def _():
            colmin_s[...] = jnp.full((1, N), BIG, jnp.float32)

        z = z_ref[0]
        px = ax_ref[0] * z
        py = ay_ref[0] * z
        mp = (mp_ref[0] > 0) & (px + py + z != 0.0)
        px = jnp.where(mp, px, SENT)
        py = jnp.where(mp, py, SENT)
        pz = jnp.where(mp, z, SENT)

        gq = gt_ref[0]
        gx, gy, gz = gq[0:1], gq[1:2], gq[2:3]
        mq = (mr_ref[0] > 0) & (gx + gy + gz != 0.0)
        gx = jnp.where(mq, gx, SENT)
        gy = jnp.where(mq, gy, SENT)
        gz = jnp.where(mq, gz, SENT)

        pp = px * px + py * py + pz * pz
        qq = gx * gx + gy * gy + gz * gz
        one_p = jnp.ones((1, RB), jnp.float32)
        one_q = jnp.ones((1, N), jnp.float32)
        zer_p = jnp.zeros((3, RB), jnp.float32)
        zer_q = jnp.zeros((3, N), jnp.float32)

        pm = jnp.concatenate(
            [-2.0 * px, -2.0 * py, -2.0 * pz, pp, one_p, zer_p], axis=0)
        qm = jnp.concatenate([gx, gy, gz, one_q, qq, zer_q], axis=0)
        d = lax.dot_general(jnp.transpose(pm, (1, 0)), qm,
                            (((1,), (0,)), ((), ())),
                            preferred_element_type=jnp.float32)
        d = jnp.maximum(d, 0.0)

        rowmin = jnp.minimum(jnp.min(d, axis=1, keepdims=True), BIG)
        mpf = jnp.where(mp, 1.0, 0.0)
        s1 = lax.dot_general(mpf, rowmin, (((1,), (0,)), ((), ())),
                             preferred_element_type=jnp.float32)
        acc_s[0, 0] += s1[0, 0]

        colmin_s[...] = jnp.minimum(colmin_s[...],
                                    jnp.min(d, axis=0, keepdims=True))

        @pl.when(r == NRB - 1)
        def _():
            out2_ref[...] = jnp.minimum(colmin_s[...], BIG)[0]

        @pl.when((b == B - 1) & (r == NRB - 1))
        def _():
            out1_ref[0, 0] = acc_s[0, 0]

    return pl.pallas_call(
        body,
        grid=(B, NRB),
        in_specs=[
            pl.BlockSpec((1, 1, RB), lambda b, r: (b, 0, r)),
            pl.BlockSpec((1, 1, RB), lambda b, r: (0, 0, r)),
            pl.BlockSpec((1, 1, RB), lambda b, r: (0, 0, r)),
            pl.BlockSpec((1, 1, RB), lambda b, r: (b, 0, r)),
            pl.BlockSpec((1, 3, N), lambda b, r: (b, 0, 0)),
            pl.BlockSpec((1, 1, N), lambda b, r: (b, 0, 0)),
        ],
        out_specs=[
            pl.BlockSpec(memory_space=pltpu.SMEM),
            pl.BlockSpec((N,), lambda b, r: (b,)),
        ],
        out_shape=[
            jax.ShapeDtypeStruct((1, 1), jnp.float32),
            jax.ShapeDtypeStruct((B * N,), jnp.float32),
        ],
        scratch_shapes=[pltpu.VMEM((1, N), jnp.float32),
                        pltpu.SMEM((1, 1), jnp.float32)],
    )


def _combine(B, N):
    def body(tc2_ref, sc2_ref, gx_ref, gy_ref, gz_ref, m_ref, out_ref,
             acc_s):
        b = pl.program_id(0)

        @pl.when(b == 0)
        def _():
            acc_s[0, 0] = 0.0

        gx = gx_ref[...]
        gy = gy_ref[...]
        gz = gz_ref[...]
        mq = (m_ref[...] > 0) & (gx + gy + gz != 0.0)
        cm = jnp.minimum(tc2_ref[...], sc2_ref[...])
        acc_s[0, 0] += jnp.sum(jnp.where(mq, cm, 0.0))

        @pl.when(b == B - 1)
        def _():
            out_ref[0, 0] = acc_s[0, 0]

    vspec = pl.BlockSpec((N,), lambda b: (b,))
    return pl.pallas_call(
        body,
        grid=(B,),
        in_specs=[vspec] * 6,
        out_specs=pl.BlockSpec(memory_space=pltpu.SMEM),
        out_shape=jax.ShapeDtypeStruct((1, 1), jnp.float32),
        scratch_shapes=[pltpu.SMEM((1, 1), jnp.float32)],
    )


def kernel(pred, gt_xyz, mask, fx, fy, cx, cy):
    B, _, H, W = pred.shape
    N = H * W
    fx = jnp.asarray(fx, jnp.float32)
    fy = jnp.asarray(fy, jnp.float32)
    cx = jnp.asarray(cx, jnp.float32)
    cy = jnp.asarray(cy, jnp.float32)

    z = pred.reshape(B * N).astype(jnp.float32)
    gxf = gt_xyz[:, 0, :, :].reshape(B * N).astype(jnp.float32)
    gyf = gt_xyz[:, 1, :, :].reshape(B * N).astype(jnp.float32)
    gzf = gt_xyz[:, 2, :, :].reshape(B * N).astype(jnp.float32)
    m = mask.reshape(B * N).astype(jnp.int32)
    n = jnp.arange(N, dtype=jnp.int32)
    ax = ((n % W).astype(jnp.float32) - cx) / fx
    ay = ((n // W).astype(jnp.float32) - cy) / fy

    sc1, sc2 = _sc_chamfer(B, N)(z, ax, ay, gxf, gyf, gzf, m)

    tc1, tc2 = _tc_chamfer(B, N)(
        pred.reshape(B, 1, N).astype(jnp.float32),
        ax.reshape(1, 1, N), ay.reshape(1, 1, N),
        mask.reshape(B, 1, N).astype(jnp.int32),
        gt_xyz.reshape(B, 3, N).astype(jnp.float32),
        mask.reshape(B, 1, N).astype(jnp.int32))

    d2 = _combine(B, N)(tc2, sc2, gxf, gyf, gzf, m)

    total = jnp.sum(sc1) + tc1[0, 0] + d2[0, 0]
    return total / jnp.float32(B)


# speedup vs baseline: 2.3489x; 1.0270x over previous
import functools

import jax
import jax.numpy as jnp
from jax import lax
from jax.experimental import pallas as pl
from jax.experimental.pallas import tpu as pltpu
from jax.experimental.pallas import tpu_sc as plsc

L = 16
NW = 32
SLOTS = 8
BIG = 1e10
SENT = 1e5
PAD = 2 * L
R_SPLIT = 3328
RB = 256


def _sc_chamfer(B, N):
    mesh = plsc.VectorSubcoreMesh(core_axis_name="c", subcore_axis_name="s")
    NP = N - R_SPLIT
    QS = N // SLOTS

    @functools.partial(
        pl.kernel,
        mesh=mesh,
        out_type=(jax.ShapeDtypeStruct((NW * L,), jnp.float32),
                  jax.ShapeDtypeStruct((B * N,), jnp.float32)),
        scratch_types=[
            pltpu.VMEM((NP,), jnp.float32),
            pltpu.VMEM((NP,), jnp.float32),
            pltpu.VMEM((NP,), jnp.float32),
            pltpu.VMEM((N,), jnp.float32),
            pltpu.VMEM((N,), jnp.float32),
            pltpu.VMEM((N,), jnp.float32),
            pltpu.VMEM((N,), jnp.int32),
            pltpu.VMEM((NP + PAD,), jnp.float32),
            pltpu.VMEM((NP + PAD,), jnp.float32),
            pltpu.VMEM((NP + PAD,), jnp.float32),
            pltpu.VMEM((N + PAD,), jnp.float32),
            pltpu.VMEM((N + PAD,), jnp.float32),
            pltpu.VMEM((N + PAD,), jnp.float32),
            pltpu.VMEM((L,), jnp.float32),
            pltpu.VMEM((QS,), jnp.float32),
        ],
        compiler_params=pltpu.CompilerParams(needs_layout_passes=False),
    )
    def cham(z_hbm, ax_hbm, ay_hbm, gx_hbm, gy_hbm, gz_hbm, m_hbm,
             out1_hbm, out2_hbm,
             z_v, ax_v, ay_v, gx_v, gy_v, gz_v, m_v,
             cpx, cpy, cpz, cgx, cgy, cgz, acc_v, minb_v):
        cid = lax.axis_index("c")
        sid = lax.axis_index("s")
        wid = sid * 2 + cid
        bat = wid // SLOTS
        slot = wid % SLOTS
        boff = bat * N

        pltpu.sync_copy(z_hbm.at[pl.ds(boff + R_SPLIT, NP)], z_v)
        pltpu.sync_copy(ax_hbm.at[pl.ds(R_SPLIT, NP)], ax_v)
        pltpu.sync_copy(ay_hbm.at[pl.ds(R_SPLIT, NP)], ay_v)
        pltpu.sync_copy(gx_hbm.at[pl.ds(boff, N)], gx_v)
        pltpu.sync_copy(gy_hbm.at[pl.ds(boff, N)], gy_v)
        pltpu.sync_copy(gz_hbm.at[pl.ds(boff, N)], gz_v)
        pltpu.sync_copy(m_hbm.at[pl.ds(boff, N)], m_v)

        def comp_g(i, n_g):
            sl = pl.ds(i * L, L)
            gxc = gx_v[sl]
            gyc = gy_v[sl]
            gzc = gz_v[sl]
            mg = (m_v[sl] > 0) & (gxc + gyc + gzc != 0.0)
            mgi = mg.astype(jnp.int32)
            gidx = n_g + (plsc.cumsum(mgi) - mgi)
            plsc.store_scatter(cgx, [gidx], gxc, mask=mg)
            plsc.store_scatter(cgy, [gidx], gyc, mask=mg)
            plsc.store_scatter(cgz, [gidx], gzc, mask=mg)
            return n_g + jnp.sum(mgi)

        def comp_p(i, n_p):
            sl = pl.ds(i * L, L)
            zc = z_v[sl]
            pxc = ax_v[sl] * zc
            pyc = ay_v[sl] * zc
            mp = (m_v[sl] > 0) & (pxc + pyc + zc != 0.0)
            mpi = mp.astype(jnp.int32)
            pidx = n_p + (plsc.cumsum(mpi) - mpi)
            plsc.store_scatter(cpx, [pidx], pxc, mask=mp)
            plsc.store_scatter(cpy, [pidx], pyc, mask=mp)
            plsc.store_scatter(cpz, [pidx], zc, mask=mp)
            return n_p + jnp.sum(mpi)

        n_g = lax.fori_loop(0, N // L, comp_g, jnp.int32(0))
        n_p = lax.fori_loop(0, NP // L, comp_p, jnp.int32(0))

        sent = jnp.full((L,), SENT, jnp.float32)
        cpx[pl.ds(n_p, L)] = sent
        cpy[pl.ds(n_p, L)] = sent
        cpz[pl.ds(n_p, L)] = sent
        cgx[pl.ds(n_g, L)] = sent
        cgy[pl.ds(n_g, L)] = sent
        cgz[pl.ds(n_g, L)] = sent

        lane = lax.iota(jnp.int32, L)

        def nn_min(qx, qy, qz, rx_r, ry_r, rz_r, nr8):
            def rloop(j, rmin):
                j8 = j * 8
                for u in range(8):
                    jv = jnp.full((L,), j8 + u, jnp.int32)
                    rx = plsc.load_gather(rx_r, [jv])
                    ry = plsc.load_gather(ry_r, [jv])
                    rz = plsc.load_gather(rz_r, [jv])
                    dx = qx - rx
                    dy = qy - ry
                    dz = qz - rz
                    d = dx * dx + dy * dy + dz * dz
                    rmin = jnp.minimum(rmin, d)
                return rmin

            return lax.fori_loop(0, nr8, rloop,
                                 jnp.full((L,), BIG, jnp.float32))

        qper = (n_p + SLOTS - 1) // SLOTS
        qlo = slot * qper
        qhi = jnp.minimum(n_p, qlo + qper)
        nblk = (jnp.maximum(0, qhi - qlo) + L - 1) // L
        ng8 = (n_g + 7) // 8

        def qblock(ib, acc):
            base = qlo + ib * L
            rmin = nn_min(cpx[pl.ds(base, L)], cpy[pl.ds(base, L)],
                          cpz[pl.ds(base, L)], cgx, cgy, cgz, ng8)
            valid = (base + lane) < qhi
            return acc + jnp.where(valid, rmin, 0.0)

        acc = lax.fori_loop(0, nblk, qblock, jnp.zeros((L,), jnp.float32))
        acc_v[...] = acc
        pltpu.sync_copy(acc_v, out1_hbm.at[pl.ds(wid * L, L)])

        np8 = (n_p + 7) // 8
        gbase = slot * QS

        def gblock(ib, _):
            off = ib * L
            sl = pl.ds(gbase + off, L)
            rmin = nn_min(gx_v[sl], gy_v[sl], gz_v[sl],
                          cpx, cpy, cpz, np8)
            minb_v[pl.ds(off, L)] = rmin
            return 0

        lax.fori_loop(0, QS // L, gblock, 0)
        pltpu.sync_copy(minb_v, out2_hbm.at[pl.ds(boff + gbase, QS)])

    return cham


def _tc_chamfer(B, N):
    NRB = R_SPLIT // RB

    def body(z_ref, ax_ref, ay_ref, mp_ref, gt_ref, mr_ref,
             out1_ref, out2_ref, colmin_s, acc_s):
        b = pl.program_id(0)
        r = pl.program_id(1)

        @pl.when((b == 0) & (r == 0))
        def _():
            acc_s[0, 0] = 0.0

        @pl.when(r == 0)
        def _():
            colmin_s[...] = jnp.full((1, N), BIG, jnp.float32)

        z = z_ref[0]
        px = ax_ref[0] * z
        py = ay_ref[0] * z
        mp = (mp_ref[0] > 0) & (px + py + z != 0.0)
        px = jnp.where(mp, px, SENT)
        py = jnp.where(mp, py, SENT)
        pz = jnp.where(mp, z, SENT)

        gq = gt_ref[0]
        gx, gy, gz = gq[0:1], gq[1:2], gq[2:3]
        mq = (mr_ref[0] > 0) & (gx + gy + gz != 0.0)
        gx = jnp.where(mq, gx, SENT)
        gy = jnp.where(mq, gy, SENT)
        gz = jnp.where(mq, gz, SENT)

        pp = px * px + py * py + pz * pz
        qq = gx * gx + gy * gy + gz * gz
        one_p = jnp.ones((1, RB), jnp.float32)
        one_q = jnp.ones((1, N), jnp.float32)
        zer_p = jnp.zeros((3, RB), jnp.float32)
        zer_q = jnp.zeros((3, N), jnp.float32)

        pm = jnp.concatenate(
            [-2.0 * px, -2.0 * py, -2.0 * pz, pp, one_p, zer_p], axis=0)
        qm = jnp.concatenate([gx, gy, gz, one_q, qq, zer_q], axis=0)
        d = lax.dot_general(jnp.transpose(pm, (1, 0)), qm,
                            (((1,), (0,)), ((), ())),
                            preferred_element_type=jnp.float32)
        d = jnp.maximum(d, 0.0)

        rowmin = jnp.minimum(jnp.min(d, axis=1, keepdims=True), BIG)
        mpf = jnp.where(mp, 1.0, 0.0)
        s1 = lax.dot_general(mpf, rowmin, (((1,), (0,)), ((), ())),
                             preferred_element_type=jnp.float32)
        acc_s[0, 0] += s1[0, 0]

        colmin_s[...] = jnp.minimum(colmin_s[...],
                                    jnp.min(d, axis=0, keepdims=True))

        @pl.when(r == NRB - 1)
        def _():
            out2_ref[...] = jnp.minimum(colmin_s[...], BIG)[0]

        @pl.when((b == B - 1) & (r == NRB - 1))
        def _():
            out1_ref[0, 0] = acc_s[0, 0]

    return pl.pallas_call(
        body,
        grid=(B, NRB),
        in_specs=[
            pl.BlockSpec((1, 1, RB), lambda b, r: (b, 0, r)),
            pl.BlockSpec((1, 1, RB), lambda b, r: (0, 0, r)),
            pl.BlockSpec((1, 1, RB), lambda b, r: (0, 0, r)),
            pl.BlockSpec((1, 1, RB), lambda b, r: (b, 0, r)),
            pl.BlockSpec((1, 3, N), lambda b, r: (b, 0, 0)),
            pl.BlockSpec((1, 1, N), lambda b, r: (b, 0, 0)),
        ],
        out_specs=[
            pl.BlockSpec(memory_space=pltpu.SMEM),
            pl.BlockSpec((N,), lambda b, r: (b,)),
        ],
        out_shape=[
            jax.ShapeDtypeStruct((1, 1), jnp.float32),
            jax.ShapeDtypeStruct((B * N,), jnp.float32),
        ],
        scratch_shapes=[pltpu.VMEM((1, N), jnp.float32),
                        pltpu.SMEM((1, 1), jnp.float32)],
    )


def _combine(B, N):
    def body(sc1_ref, tc2_ref, sc2_ref, gx_ref, gy_ref, gz_ref, m_ref,
             out_ref, acc_s):
        b = pl.program_id(0)

        @pl.when(b == 0)
        def _():
            acc_s[0, 0] = jnp.sum(sc1_ref[...])

        gx = gx_ref[...]
        gy = gy_ref[...]
        gz = gz_ref[...]
        mq = (m_ref[...] > 0) & (gx + gy + gz != 0.0)
        cm = jnp.minimum(tc2_ref[...], sc2_ref[...])
        acc_s[0, 0] += jnp.sum(jnp.where(mq, cm, 0.0))

        @pl.when(b == B - 1)
        def _():
            out_ref[0, 0] = acc_s[0, 0]

    vspec = pl.BlockSpec((N,), lambda b: (b,))
    return pl.pallas_call(
        body,
        grid=(B,),
        in_specs=[pl.BlockSpec((NW * L,), lambda b: (0,))] + [vspec] * 6,
        out_specs=pl.BlockSpec(memory_space=pltpu.SMEM),
        out_shape=jax.ShapeDtypeStruct((1, 1), jnp.float32),
        scratch_shapes=[pltpu.SMEM((1, 1), jnp.float32)],
    )


def kernel(pred, gt_xyz, mask, fx, fy, cx, cy):
    B, _, H, W = pred.shape
    N = H * W
    fx = jnp.asarray(fx, jnp.float32)
    fy = jnp.asarray(fy, jnp.float32)
    cx = jnp.asarray(cx, jnp.float32)
    cy = jnp.asarray(cy, jnp.float32)

    z = pred.reshape(B * N).astype(jnp.float32)
    gxf = gt_xyz[:, 0, :, :].reshape(B * N).astype(jnp.float32)
    gyf = gt_xyz[:, 1, :, :].reshape(B * N).astype(jnp.float32)
    gzf = gt_xyz[:, 2, :, :].reshape(B * N).astype(jnp.float32)
    m = mask.reshape(B * N).astype(jnp.int32)
    n = jnp.arange(N, dtype=jnp.int32)
    ax = ((n % W).astype(jnp.float32) - cx) / fx
    ay = ((n // W).astype(jnp.float32) - cy) / fy

    sc1, sc2 = _sc_chamfer(B, N)(z, ax, ay, gxf, gyf, gzf, m)

    tc1, tc2 = _tc_chamfer(B, N)(
        pred.reshape(B, 1, N).astype(jnp.float32),
        ax.reshape(1, 1, N), ay.reshape(1, 1, N),
        mask.reshape(B, 1, N).astype(jnp.int32),
        gt_xyz.reshape(B, 3, N).astype(jnp.float32),
        mask.reshape(B, 1, N).astype(jnp.int32))

    d2 = _combine(B, N)(sc1, tc2, sc2, gxf, gyf, gzf, m)

    total = tc1[0, 0] + d2[0, 0]
    return total / jnp.float32(B)
